# Initial kernel scaffold; baseline (speedup 1.0000x reference)
import jax
import jax.numpy as jnp
from jax.experimental import pallas as pl


def kernel(a_hat_p, x_p, emb0_w1, emb0_b1, emb0_w2, emb0_b2, emb1_w1, emb1_b1, emb1_w2, emb1_b2, cw1, cb1, cw2, cb2, wm, bm):
    raise NotImplementedError("write your pallas kernel here")


# speedup vs baseline: 7.6294x; 7.6294x over previous
import functools

import jax
import jax.numpy as jnp
from jax.experimental import pallas as pl
from jax.experimental.pallas import tpu as pltpu

N_REAL = 4000
NUM_CLASSES = 64


_CHUNK = 1024


def _fused_gcn_body(a_ref, x_ref, w0_ref, w1_ref, w2_ref, w3_ref, w4_ref,
                    w5_ref, bstack_ref, wm_ref, bm_ref, o_ref, g_ref,
                    *, n_real):
    n = a_ref.shape[0]
    chunk = min(_CHUNK, n)

    def a_matmul(u):
        for c in range(n // chunk):
            g_ref[:, c * chunk:(c + 1) * chunk] = jnp.dot(
                u, a_ref[:, c * chunk:(c + 1) * chunk],
                preferred_element_type=jnp.float32)

    u = jax.lax.dot_general(
        w0_ref[...], x_ref[...], (((0,), (1,)), ((), ())),
        preferred_element_type=jnp.float32).astype(jnp.bfloat16)
    a_matmul(u)
    z = jnp.maximum(g_ref[...] + bstack_ref[:, 0:1], 0.0).astype(jnp.bfloat16)

    for li, w_ref in enumerate((w1_ref, w2_ref, w3_ref, w4_ref, w5_ref)):
        u = jax.lax.dot_general(
            w_ref[...], z, (((0,), (0,)), ((), ())),
            preferred_element_type=jnp.float32).astype(jnp.bfloat16)
        a_matmul(u)
        z = jnp.maximum(g_ref[...] + bstack_ref[:, li + 1:li + 2],
                        0.0).astype(jnp.bfloat16)

    zf = z.astype(jnp.float32)
    col = jax.lax.broadcasted_iota(jnp.int32, zf.shape, 1)
    s = jnp.sum(jnp.where(col < n_real, zf, 0.0), axis=1, keepdims=True)
    mean = (s * (1.0 / n_real)).astype(jnp.bfloat16).astype(jnp.float32)

    p = wm_ref[...].astype(jnp.float32) * mean
    logits = jnp.sum(p, axis=0, keepdims=True) + bm_ref[...]
    o_ref[...] = jnp.broadcast_to(logits, o_ref.shape)


def _fused_forward(a, x, ws, bstack, wm, bm, *, n_real):
    n = a.shape[0]
    f = x.shape[1]
    h = ws[0].shape[1]
    c = wm.shape[1]
    whole = lambda shape: pl.BlockSpec(shape, lambda i: (0,) * len(shape))
    in_specs = [
        whole((n, n)),
        whole((n, f)),
        whole((f, h)),
    ] + [whole((h, h)) for _ in range(5)] + [
        whole(bstack.shape),
        whole((h, c)),
        whole((1, c)),
    ]
    out = pl.pallas_call(
        functools.partial(_fused_gcn_body, n_real=n_real),
        out_shape=jax.ShapeDtypeStruct((8, c), jnp.float32),
        grid_spec=pltpu.PrefetchScalarGridSpec(
            num_scalar_prefetch=0,
            grid=(1,),
            in_specs=in_specs,
            out_specs=whole((8, c)),
            scratch_shapes=[pltpu.VMEM((h, n), jnp.float32)],
        ),
        compiler_params=pltpu.CompilerParams(
            dimension_semantics=("arbitrary",),
            vmem_limit_bytes=100 * 1024 * 1024,
        ),
    )(a, x, *ws, bstack, wm, bm)
    return out


def kernel(a_hat_p, x_p, emb0_w1, emb0_b1, emb0_w2, emb0_b2,
           emb1_w1, emb1_b1, emb1_w2, emb1_b2,
           cw1, cb1, cw2, cb2, wm, bm):
    ws = (emb0_w1, emb0_w2, emb1_w1, emb1_w2, cw1, cw2)
    biases = (emb0_b1, emb0_b2, emb1_b1, emb1_b2, cb1, cb2)
    bstack = jnp.concatenate([b.reshape(-1, 1) for b in biases], axis=1)
    out = _fused_forward(a_hat_p, x_p, ws, bstack, wm, bm, n_real=N_REAL)
    return out[0, :NUM_CLASSES]


# speedup vs baseline: 8.2540x; 1.0819x over previous
import functools

import jax
import jax.numpy as jnp
from jax.experimental import pallas as pl
from jax.experimental.pallas import tpu as pltpu

N_REAL = 4000
NUM_CLASSES = 64


_CHUNK = 1024
_NBLK = 8


def _fused_gcn_body(a_hbm, x_ref, w0_ref, w1_ref, w2_ref, w3_ref, w4_ref,
                    w5_ref, bstack_ref, wm_ref, bm_ref, o_ref, a_vmem, g_ref,
                    sems, *, n_real):
    n = a_hbm.shape[0]
    chunk = min(_CHUNK, n)
    nblk = _NBLK if n % _NBLK == 0 and n // _NBLK >= 8 else 1
    blk = n // nblk

    def a_blk_copy(b):
        return pltpu.make_async_copy(
            a_hbm.at[pl.ds(b * blk, blk)],
            a_vmem.at[pl.ds(b * blk, blk)],
            sems.at[b])

    for b in range(nblk):
        a_blk_copy(b).start()

    def a_matmul(u):
        for c in range(n // chunk):
            g_ref[:, c * chunk:(c + 1) * chunk] = jnp.dot(
                u, a_vmem[:, c * chunk:(c + 1) * chunk],
                preferred_element_type=jnp.float32)

    u = jax.lax.dot_general(
        w0_ref[...], x_ref[...], (((0,), (1,)), ((), ())),
        preferred_element_type=jnp.float32).astype(jnp.bfloat16)
    for b in range(nblk):
        a_blk_copy(b).wait()
        ub = u[:, b * blk:(b + 1) * blk]
        for c in range(n // chunk):
            part = jnp.dot(ub, a_vmem[b * blk:(b + 1) * blk,
                                      c * chunk:(c + 1) * chunk],
                           preferred_element_type=jnp.float32)
            if b == 0:
                g_ref[:, c * chunk:(c + 1) * chunk] = part
            else:
                g_ref[:, c * chunk:(c + 1) * chunk] += part
    z = jnp.maximum(g_ref[...] + bstack_ref[:, 0:1], 0.0).astype(jnp.bfloat16)

    for li, w_ref in enumerate((w1_ref, w2_ref, w3_ref, w4_ref, w5_ref)):
        u = jax.lax.dot_general(
            w_ref[...], z, (((0,), (0,)), ((), ())),
            preferred_element_type=jnp.float32).astype(jnp.bfloat16)
        a_matmul(u)
        z = jnp.maximum(g_ref[...] + bstack_ref[:, li + 1:li + 2],
                        0.0).astype(jnp.bfloat16)

    zf = z.astype(jnp.float32)
    col = jax.lax.broadcasted_iota(jnp.int32, zf.shape, 1)
    s = jnp.sum(jnp.where(col < n_real, zf, 0.0), axis=1, keepdims=True)
    mean = (s * (1.0 / n_real)).astype(jnp.bfloat16).astype(jnp.float32)

    p = wm_ref[...].astype(jnp.float32) * mean
    logits = jnp.sum(p, axis=0, keepdims=True) + bm_ref[...]
    o_ref[...] = jnp.broadcast_to(logits, o_ref.shape)


def _fused_forward(a, x, ws, bstack, wm, bm, *, n_real):
    n = a.shape[0]
    f = x.shape[1]
    h = ws[0].shape[1]
    c = wm.shape[1]
    whole = lambda shape: pl.BlockSpec(shape, lambda i: (0,) * len(shape))
    nblk = _NBLK if n % _NBLK == 0 and n // _NBLK >= 8 else 1
    in_specs = [
        pl.BlockSpec(memory_space=pl.ANY),
        whole((n, f)),
        whole((f, h)),
    ] + [whole((h, h)) for _ in range(5)] + [
        whole(bstack.shape),
        whole((h, c)),
        whole((1, c)),
    ]
    out = pl.pallas_call(
        functools.partial(_fused_gcn_body, n_real=n_real),
        out_shape=jax.ShapeDtypeStruct((8, c), jnp.float32),
        grid_spec=pltpu.PrefetchScalarGridSpec(
            num_scalar_prefetch=0,
            grid=(1,),
            in_specs=in_specs,
            out_specs=whole((8, c)),
            scratch_shapes=[pltpu.VMEM((n, n), jnp.bfloat16),
                            pltpu.VMEM((h, n), jnp.float32),
                            pltpu.SemaphoreType.DMA((nblk,))],
        ),
        compiler_params=pltpu.CompilerParams(
            dimension_semantics=("arbitrary",),
            vmem_limit_bytes=100 * 1024 * 1024,
        ),
    )(a, x, *ws, bstack, wm, bm)
    return out


def kernel(a_hat_p, x_p, emb0_w1, emb0_b1, emb0_w2, emb0_b2,
           emb1_w1, emb1_b1, emb1_w2, emb1_b2,
           cw1, cb1, cw2, cb2, wm, bm):
    ws = (emb0_w1, emb0_w2, emb1_w1, emb1_w2, cw1, cw2)
    biases = (emb0_b1, emb0_b2, emb1_b1, emb1_b2, cb1, cb2)
    bstack = jnp.concatenate([b.reshape(-1, 1) for b in biases], axis=1)
    out = _fused_forward(a_hat_p, x_p, ws, bstack, wm, bm, n_real=N_REAL)
    return out[0, :NUM_CLASSES]


# speedup vs baseline: 8.4927x; 1.0289x over previous
import functools

import jax
import jax.numpy as jnp
from jax.experimental import pallas as pl
from jax.experimental.pallas import tpu as pltpu

N_REAL = 4000
NUM_CLASSES = 64

_CHUNK = 1024
_NBLK = 8


def _fused_gcn_body(a_hbm, x_ref, w0_ref, w1_ref, w2_ref, w3_ref, w4_ref,
                    w5_ref, b0_ref, b1_ref, b2_ref, b3_ref, b4_ref, b5_ref,
                    wm_ref, bm_ref, o_ref, a_vmem, g_ref, sems,
                    *, n_real, num_classes):
    n = a_hbm.shape[0]
    chunk = min(_CHUNK, n)
    nblk = _NBLK if n % _NBLK == 0 and n // _NBLK >= 8 else 1
    blk = n // nblk

    def a_blk_copy(b):
        return pltpu.make_async_copy(
            a_hbm.at[pl.ds(b * blk, blk)],
            a_vmem.at[pl.ds(b * blk, blk)],
            sems.at[b])

    for b in range(nblk):
        a_blk_copy(b).start()

    def bcol(b_ref):
        return jnp.transpose(b_ref[...], (1, 0))

    def a_matmul(u):
        for c in range(n // chunk):
            g_ref[:, c * chunk:(c + 1) * chunk] = jnp.dot(
                u, a_vmem[:, c * chunk:(c + 1) * chunk],
                preferred_element_type=jnp.float32)

    u = jax.lax.dot_general(
        w0_ref[...], x_ref[...], (((0,), (1,)), ((), ())),
        preferred_element_type=jnp.float32).astype(jnp.bfloat16)
    for b in range(nblk):
        a_blk_copy(b).wait()
        ub = u[:, b * blk:(b + 1) * blk]
        for c in range(n // chunk):
            part = jnp.dot(ub, a_vmem[b * blk:(b + 1) * blk,
                                      c * chunk:(c + 1) * chunk],
                           preferred_element_type=jnp.float32)
            if b == 0:
                g_ref[:, c * chunk:(c + 1) * chunk] = part
            else:
                g_ref[:, c * chunk:(c + 1) * chunk] += part
    z = jnp.maximum(g_ref[...] + bcol(b0_ref), 0.0).astype(jnp.bfloat16)

    for w_ref, b_ref in ((w1_ref, b1_ref), (w2_ref, b2_ref),
                         (w3_ref, b3_ref), (w4_ref, b4_ref),
                         (w5_ref, b5_ref)):
        u = jax.lax.dot_general(
            w_ref[...], z, (((0,), (0,)), ((), ())),
            preferred_element_type=jnp.float32).astype(jnp.bfloat16)
        a_matmul(u)
        z = jnp.maximum(g_ref[...] + bcol(b_ref), 0.0).astype(jnp.bfloat16)

    zf = z.astype(jnp.float32)
    col = jax.lax.broadcasted_iota(jnp.int32, zf.shape, 1)
    s = jnp.sum(jnp.where(col < n_real, zf, 0.0), axis=1, keepdims=True)
    mean = (s * (1.0 / n_real)).astype(jnp.bfloat16).astype(jnp.float32)

    p = wm_ref[...].astype(jnp.float32) * mean
    logits = jnp.sum(p, axis=0, keepdims=True) + bm_ref[...]
    o_ref[...] = jnp.broadcast_to(logits, o_ref.shape)


def _fused_forward(a, x, ws, bs, wm, bm, *, n_real, num_classes):
    n = a.shape[0]
    f = x.shape[1]
    h = ws[0].shape[1]
    c = wm.shape[1]
    whole = lambda shape: pl.BlockSpec(shape, lambda i: (0,) * len(shape))
    nblk = _NBLK if n % _NBLK == 0 and n // _NBLK >= 8 else 1
    in_specs = [
        pl.BlockSpec(memory_space=pl.ANY),
        whole((n, f)),
        whole((f, h)),
    ] + [whole((h, h)) for _ in range(5)] + [
        whole((1, h)) for _ in range(6)
    ] + [
        whole((h, c)),
        whole((1, c)),
    ]
    out = pl.pallas_call(
        functools.partial(_fused_gcn_body, n_real=n_real,
                          num_classes=num_classes),
        out_shape=jax.ShapeDtypeStruct((8, c), jnp.float32),
        grid_spec=pltpu.PrefetchScalarGridSpec(
            num_scalar_prefetch=0,
            grid=(1,),
            in_specs=in_specs,
            out_specs=whole((8, c)),
            scratch_shapes=[pltpu.VMEM((n, n), jnp.bfloat16),
                            pltpu.VMEM((h, n), jnp.float32),
                            pltpu.SemaphoreType.DMA((nblk,))],
        ),
        compiler_params=pltpu.CompilerParams(
            dimension_semantics=("arbitrary",),
            vmem_limit_bytes=100 * 1024 * 1024,
        ),
    )(a, x, *ws, *bs, wm, bm)
    return out


def kernel(a_hat_p, x_p, emb0_w1, emb0_b1, emb0_w2, emb0_b2,
           emb1_w1, emb1_b1, emb1_w2, emb1_b2,
           cw1, cb1, cw2, cb2, wm, bm):
    ws = (emb0_w1, emb0_w2, emb1_w1, emb1_w2, cw1, cw2)
    bs = (emb0_b1, emb0_b2, emb1_b1, emb1_b2, cb1, cb2)
    out = _fused_forward(a_hat_p, x_p, ws, bs, wm, bm,
                         n_real=N_REAL, num_classes=NUM_CLASSES)
    return out[0, :NUM_CLASSES]


# speedup vs baseline: 8.7872x; 1.0347x over previous
import functools

import jax
import jax.numpy as jnp
from jax.experimental import pallas as pl
from jax.experimental.pallas import tpu as pltpu

N_REAL = 4000
NUM_CLASSES = 64

_CHUNK = 1024
_NBLK = 8


def _fused_gcn_body(a_hbm, x_ref, w0_ref, w1_ref, w2_ref, w3_ref, w4_ref,
                    w5_ref, b0_ref, b1_ref, b2_ref, b3_ref, b4_ref, b5_ref,
                    wm_ref, bm_ref, o_ref, a_vmem, g_ref, sems,
                    *, n_real, num_classes):
    n = a_hbm.shape[0]
    chunk = min(_CHUNK, n)
    nblk = _NBLK if n % _NBLK == 0 and n // _NBLK >= 8 else 1
    blk = n // nblk

    def a_blk_copy(b):
        return pltpu.make_async_copy(
            a_hbm.at[pl.ds(b * blk, blk)],
            a_vmem.at[pl.ds(b * blk, blk)],
            sems.at[b])

    for b in range(nblk):
        a_blk_copy(b).start()

    def bcol(b_ref):
        return jnp.transpose(b_ref[...], (1, 0))

    def a_matmul(u):
        for c in range(n // chunk):
            g_ref[:, c * chunk:(c + 1) * chunk] = jnp.dot(
                u, a_vmem[:, c * chunk:(c + 1) * chunk],
                preferred_element_type=jnp.float32)

    u = jax.lax.dot_general(
        w0_ref[...], x_ref[...], (((0,), (1,)), ((), ())),
        preferred_element_type=jnp.float32).astype(jnp.bfloat16)
    for b in range(nblk):
        a_blk_copy(b).wait()
        ub = u[:, b * blk:(b + 1) * blk]
        for c in range(n // chunk):
            part = jnp.dot(ub, a_vmem[b * blk:(b + 1) * blk,
                                      c * chunk:(c + 1) * chunk],
                           preferred_element_type=jnp.float32)
            if b == 0:
                g_ref[:, c * chunk:(c + 1) * chunk] = part
            else:
                g_ref[:, c * chunk:(c + 1) * chunk] += part
    z = jnp.maximum(g_ref[...] + bcol(b0_ref), 0.0).astype(jnp.bfloat16)

    for w_ref, b_ref in ((w1_ref, b1_ref), (w2_ref, b2_ref),
                         (w3_ref, b3_ref), (w4_ref, b4_ref),
                         (w5_ref, b5_ref)):
        u = jax.lax.dot_general(
            w_ref[...], z, (((0,), (0,)), ((), ())),
            preferred_element_type=jnp.float32).astype(jnp.bfloat16)
        a_matmul(u)
        z = jnp.maximum(g_ref[...] + bcol(b_ref), 0.0).astype(jnp.bfloat16)

    zf = z.astype(jnp.float32)
    col = jax.lax.broadcasted_iota(jnp.int32, zf.shape, 1)
    s = jnp.sum(jnp.where(col < n_real, zf, 0.0), axis=1, keepdims=True)
    mean = (s * (1.0 / n_real)).astype(jnp.bfloat16).astype(jnp.float32)

    p = wm_ref[...].astype(jnp.float32) * mean
    logits = jnp.sum(p, axis=0, keepdims=True) + bm_ref[...]
    o_ref[...] = logits[0, :num_classes]


def _fused_forward(a, x, ws, bs, wm, bm, *, n_real, num_classes):
    n = a.shape[0]
    f = x.shape[1]
    h = ws[0].shape[1]
    c = wm.shape[1]
    whole = lambda shape: pl.BlockSpec(shape, lambda i: (0,) * len(shape))
    nblk = _NBLK if n % _NBLK == 0 and n // _NBLK >= 8 else 1
    in_specs = [
        pl.BlockSpec(memory_space=pl.ANY),
        whole((n, f)),
        whole((f, h)),
    ] + [whole((h, h)) for _ in range(5)] + [
        whole((1, h)) for _ in range(6)
    ] + [
        whole((h, c)),
        whole((1, c)),
    ]
    out = pl.pallas_call(
        functools.partial(_fused_gcn_body, n_real=n_real,
                          num_classes=num_classes),
        out_shape=jax.ShapeDtypeStruct((num_classes,), jnp.float32),
        grid_spec=pltpu.PrefetchScalarGridSpec(
            num_scalar_prefetch=0,
            grid=(1,),
            in_specs=in_specs,
            out_specs=whole((num_classes,)),
            scratch_shapes=[pltpu.VMEM((n, n), jnp.bfloat16),
                            pltpu.VMEM((h, n), jnp.float32),
                            pltpu.SemaphoreType.DMA((nblk,))],
        ),
        compiler_params=pltpu.CompilerParams(
            dimension_semantics=("arbitrary",),
            vmem_limit_bytes=100 * 1024 * 1024,
        ),
    )(a, x, *ws, *bs, wm, bm)
    return out


def kernel(a_hat_p, x_p, emb0_w1, emb0_b1, emb0_w2, emb0_b2,
           emb1_w1, emb1_b1, emb1_w2, emb1_b2,
           cw1, cb1, cw2, cb2, wm, bm):
    ws = (emb0_w1, emb0_w2, emb1_w1, emb1_w2, cw1, cw2)
    bs = (emb0_b1, emb0_b2, emb1_b1, emb1_b2, cb1, cb2)
    return _fused_forward(a_hat_p, x_p, ws, bs, wm, bm,
                          n_real=N_REAL, num_classes=NUM_CLASSES)
